# Initial kernel scaffold; baseline (speedup 1.0000x reference)
import jax
import jax.numpy as jnp
from jax.experimental import pallas as pl


def kernel(local_entity, batch_heads, batch_rels, batch_tails, batch_ids, fact_ids, weight_list, weight_rel_list, rel_features, W, b):
    raise NotImplementedError("write your pallas kernel here")


# speedup vs baseline: 2.2140x; 2.2140x over previous
import functools

import jax
import jax.numpy as jnp
from jax import lax
from jax.experimental import pallas as pl
from jax.experimental.pallas import tpu as pltpu
from jax.experimental.pallas import tpu_sc as plsc

_B, _L, _E, _R, _D = 10, 1000, 320000, 512, 128
_N_ENT = _B * _L

_N_CORES = 2
_N_TILES = 16
_N_RANGES = 4
_ROWS = _N_ENT // _N_RANGES
_ACC_WORDS = _ROWS * _R
_ACC_PAD = _ACC_WORDS + 2048
_ZSLICE = _ACC_PAD // _N_TILES
_WSLICE = _ACC_WORDS // _N_TILES
_DUMMY = _ACC_WORDS

_CHUNK = 2048
_SUBROWS = _CHUNK // 128
_Q = 20480
_E_PAD = _N_TILES * _Q


_sc_mesh = plsc.VectorSubcoreMesh(core_axis_name="c", subcore_axis_name="s")


@functools.partial(
    pl.kernel,
    out_type=jax.ShapeDtypeStruct((_N_RANGES, _ACC_WORDS), jnp.float32),
    mesh=_sc_mesh,
    scratch_types=[
        pltpu.VMEM((_CHUNK,), jnp.int32),
        pltpu.VMEM((_CHUNK,), jnp.int32),
        pltpu.VMEM((_CHUNK,), jnp.int32),
        pltpu.VMEM((_SUBROWS, 128), jnp.float32),
        pltpu.VMEM((_SUBROWS, 128), jnp.int32),
        pltpu.VMEM((_SUBROWS, 128), jnp.int32),
        pltpu.VMEM_SHARED((_ACC_PAD,), jnp.float32),
    ],
)
def _build_c(tails, heads, rels, w2d, zeros_hbm, out,
             t_buf, h_buf, r_buf, w_buf, ft_buf, fh_buf, acc):
    c = lax.axis_index("c")
    s = lax.axis_index("s")

    for rng in range(_N_RANGES // _N_CORES):
        rid = c * (_N_RANGES // _N_CORES) + rng
        base_row = rid * _ROWS

        pltpu.sync_copy(zeros_hbm,
                        acc.at[pl.ds(pl.multiple_of(s * _ZSLICE, 128), _ZSLICE)])
        plsc.subcore_barrier()

        def chunk_body(k, _):
            off = s * _Q + k * _CHUNK
            pltpu.sync_copy(tails.at[pl.ds(off, _CHUNK)], t_buf)
            pltpu.sync_copy(heads.at[pl.ds(off, _CHUNK)], h_buf)
            pltpu.sync_copy(rels.at[pl.ds(off, _CHUNK)], r_buf)
            row_off = pl.multiple_of(off // 128, 8)
            pltpu.sync_copy(w2d.at[pl.ds(row_off, _SUBROWS)], w_buf)

            def vec_body(i, _):
                j = i // 8
                l = i - j * 8
                sl = pl.ds(i * 16, 16)
                dsl = pl.ds(l * 16, 16)
                tv = t_buf[sl]
                hv = h_buf[sl]
                rv = r_buf[sl]
                lt = tv - base_row
                ft = jnp.where((lt >= 0) & (lt < _ROWS), lt * _R + rv, _DUMMY)
                lh = hv - base_row
                fh = jnp.where((lh >= 0) & (lh < _ROWS), lh * _R + rv, _DUMMY)
                ft_buf[j, dsl] = ft
                fh_buf[j, dsl] = fh
                return 0

            lax.fori_loop(0, _CHUNK // 16, vec_body, 0)

            def scat_body(j, _):
                pltpu.sync_copy(w_buf.at[j], acc.at[ft_buf.at[j]], add=True)
                pltpu.sync_copy(w_buf.at[j], acc.at[fh_buf.at[j]], add=True)
                return 0

            lax.fori_loop(0, _SUBROWS, scat_body, 0)
            return 0

        lax.fori_loop(0, _Q // _CHUNK, chunk_body, 0)
        plsc.subcore_barrier()

        woff = pl.multiple_of(s * _WSLICE, 128)
        pltpu.sync_copy(acc.at[pl.ds(woff, _WSLICE)],
                        out.at[rid, pl.ds(woff, _WSLICE)])
        plsc.subcore_barrier()


def _relval_body(rf_ref, wt_ref, b_ref, o_ref):
    o_ref[...] = (
        jnp.dot(rf_ref[...], wt_ref[...], preferred_element_type=jnp.float32)
        + b_ref[...]
    )


def _mm_relu_body(c_ref, rv_ref, o_ref):
    o_ref[...] = jnp.maximum(
        jnp.dot(c_ref[...], rv_ref[...], preferred_element_type=jnp.float32),
        0.0,
    )


def kernel(local_entity, batch_heads, batch_rels, batch_tails, batch_ids,
           fact_ids, weight_list, weight_rel_list, rel_features, W, b):
    del local_entity, batch_ids, fact_ids, weight_list

    pad = _E_PAD - _E
    zi = jnp.zeros((pad,), jnp.int32)
    tails = jnp.concatenate([batch_tails.astype(jnp.int32), zi])
    heads = jnp.concatenate([batch_heads.astype(jnp.int32), zi])
    rels = jnp.concatenate([batch_rels.astype(jnp.int32), zi])
    w2d = jnp.concatenate(
        [weight_rel_list.astype(jnp.float32), jnp.zeros((pad,), jnp.float32)]
    ).reshape(_E_PAD // 128, 128)
    zeros_hbm = jnp.zeros((_ZSLICE,), jnp.float32)

    c_flat = _build_c(tails, heads, rels, w2d, zeros_hbm)
    C = c_flat.reshape(_N_ENT, _R)

    rel_val = pl.pallas_call(
        _relval_body,
        out_shape=jax.ShapeDtypeStruct((_R, _D), jnp.float32),
    )(rel_features.astype(jnp.float32), W.astype(jnp.float32).T,
      b.astype(jnp.float32).reshape(1, _D))

    rows_blk = 2000
    out = pl.pallas_call(
        _mm_relu_body,
        grid=(_N_ENT // rows_blk,),
        in_specs=[
            pl.BlockSpec((rows_blk, _R), lambda i: (i, 0)),
            pl.BlockSpec((_R, _D), lambda i: (0, 0)),
        ],
        out_specs=pl.BlockSpec((rows_blk, _D), lambda i: (i, 0)),
        out_shape=jax.ShapeDtypeStruct((_N_ENT, _D), jnp.float32),
    )(C, rel_val)

    return out.reshape(_B, _L, _D)
